# Initial kernel scaffold; baseline (speedup 1.0000x reference)
import jax
import jax.numpy as jnp
from jax.experimental import pallas as pl


def kernel(inputs, lookup_table):
    raise NotImplementedError("write your pallas kernel here")


# speedup vs baseline: 1.0735x; 1.0735x over previous
import functools

import jax
import jax.numpy as jnp
from jax import lax
from jax.experimental import pallas as pl
from jax.experimental.pallas import tpu as pltpu
from jax.experimental.pallas import tpu_sc as plsc

_NC = 2
_NS = 16
_L = 16
_NW = _NC * _NS


@functools.lru_cache(maxsize=None)
def _build(B, V, D, chunk):
    assert D == 2 * _L
    per_w = B // _NW
    assert per_w * _NW == B
    chunks = per_w // chunk
    assert chunks * chunk == per_w
    scale = jnp.float32(float(D) ** 0.5)

    mesh = plsc.VectorSubcoreMesh(core_axis_name="c", subcore_axis_name="s")

    @functools.partial(
        pl.kernel,
        mesh=mesh,
        out_type=jax.ShapeDtypeStruct((B, D), jnp.float32),
        compiler_params=pltpu.CompilerParams(
            needs_layout_passes=False, use_tc_tiling_on_sc=False
        ),
        scratch_types=[
            pltpu.VMEM((per_w,), jnp.int32),
            pltpu.VMEM((chunk, D), jnp.float32),
            pltpu.VMEM((chunk, D), jnp.float32),
            pltpu.SemaphoreType.DMA,
            pltpu.SemaphoreType.DMA,
        ],
    )
    def k(idx_hbm, table_hbm, out_hbm, idx_v, rows0, rows1, sem0, sem1):
        wid = lax.axis_index("s") * _NC + lax.axis_index("c")
        base = wid * per_w
        pltpu.sync_copy(idx_hbm.at[pl.ds(base, per_w)], idx_v)

        rows = (rows0, rows1)
        sems = (sem0, sem1)

        def start(g):
            b = g % 2
            return pltpu.async_copy(
                table_hbm.at[idx_v.at[pl.ds(g * chunk, chunk)]], rows[b], sems[b]
            )

        pending = start(0)
        for g in range(chunks):
            rv = rows[g % 2]
            pending.wait()
            if g + 1 < chunks:
                pending = start(g + 1)

            def srow(r, _):
                rv[r, pl.ds(0, _L)] = rv[r, pl.ds(0, _L)] * scale
                rv[r, pl.ds(_L, _L)] = rv[r, pl.ds(_L, _L)] * scale
                return 0

            lax.fori_loop(0, chunk, srow, 0)

            def zrow(j, _):
                iv = idx_v[pl.ds(g * chunk + j * _L, _L)]
                zmask = iv == 0

                nz = plsc.all_reduce_population_count(zmask)

                @pl.when(nz[0] > 0)
                def _():
                    row_ids = j * _L + lax.iota(jnp.int32, _L)

                    def zcol(c, _):
                        plsc.store_scatter(
                            rv,
                            [row_ids, jnp.full((_L,), c, jnp.int32)],
                            jnp.zeros((_L,), jnp.float32),
                            mask=zmask,
                        )
                        return 0

                    lax.fori_loop(0, D, zcol, 0)

                return 0

            lax.fori_loop(0, chunk // _L, zrow, 0)
            pltpu.sync_copy(rv, out_hbm.at[pl.ds(base + g * chunk, chunk)])

    return k


def kernel(inputs, lookup_table):
    V, D = lookup_table.shape
    B = inputs.size
    idx = inputs.reshape(-1).astype(jnp.int32)
    out = _build(B, V, D, 1280)(idx, lookup_table)
    return out.reshape(*inputs.shape, D)


# speedup vs baseline: 1.5569x; 1.4503x over previous
import functools

import jax
import jax.numpy as jnp
from jax import lax
from jax.experimental import pallas as pl
from jax.experimental.pallas import tpu as pltpu
from jax.experimental.pallas import tpu_sc as plsc

_NC = 2
_NS = 16
_L = 16
_NW = _NC * _NS


@functools.lru_cache(maxsize=None)
def _build(N, I, V, D, K):
    assert D == 2 * _L
    B = N * I
    per_w = B // _NW
    assert per_w * _NW == B
    units_i = I // K
    assert units_i * K == I
    units = N * units_i
    per_w_units = units // _NW
    assert per_w_units * _NW == units
    scale = jnp.float32(float(D) ** 0.5)

    mesh = plsc.VectorSubcoreMesh(core_axis_name="c", subcore_axis_name="s")

    @functools.partial(
        pl.kernel,
        mesh=mesh,
        out_type=jax.ShapeDtypeStruct((N, D, I), jnp.float32),
        compiler_params=pltpu.CompilerParams(
            needs_layout_passes=False, use_tc_tiling_on_sc=False
        ),
        scratch_types=[
            pltpu.VMEM((per_w,), jnp.int32),
            pltpu.VMEM((K, D), jnp.float32),
            pltpu.VMEM((K, D), jnp.float32),
            pltpu.VMEM((D, K), jnp.float32),
            pltpu.VMEM((D, K), jnp.float32),
            pltpu.SemaphoreType.DMA,
            pltpu.SemaphoreType.DMA,
            pltpu.SemaphoreType.DMA,
            pltpu.SemaphoreType.DMA,
        ],
    )
    def k(idx_hbm, table_hbm, out_hbm, idx_v, raw0, raw1, tr0, tr1,
          gsem0, gsem1, wsem0, wsem1):
        wid = lax.axis_index("s") * _NC + lax.axis_index("c")
        base_u = wid * per_w_units
        pltpu.sync_copy(idx_hbm.at[pl.ds(wid * per_w, per_w)], idx_v)

        raws = (raw0, raw1)
        trs = (tr0, tr1)
        gsems = (gsem0, gsem1)
        wsems = (wsem0, wsem1)

        def start_gather(t):
            return pltpu.async_copy(
                table_hbm.at[idx_v.at[pl.ds(t * K, K)]], raws[t % 2], gsems[t % 2]
            )

        lane = lax.iota(jnp.int32, _L)
        u_lo = lane
        u_hi = _L + lane

        pending_g = start_gather(0)
        pending_w = [None, None]
        for t in range(per_w_units):
            u = base_u + t
            j = u // units_i
            i0 = (u % units_i) * K
            raw = raws[t % 2]
            tr = trs[t % 2]

            pending_g.wait()
            if t + 1 < per_w_units:
                pending_g = start_gather(t + 1)
            if pending_w[t % 2] is not None:
                pending_w[t % 2].wait()

            def trow(r, _):
                rcol = jnp.full((_L,), r, jnp.int32)
                plsc.store_scatter(tr, [u_lo, rcol], raw[r, pl.ds(0, _L)] * scale)
                plsc.store_scatter(tr, [u_hi, rcol], raw[r, pl.ds(_L, _L)] * scale)
                return 0

            lax.fori_loop(0, K, trow, 0, unroll=4)

            def zgrp(g, _):
                iv = idx_v[pl.ds(t * K + g * _L, _L)]
                zmask = iv == 0
                nz = plsc.all_reduce_population_count(zmask)

                @pl.when(nz[0] > 0)
                def _():
                    cols = g * _L + lane

                    def zrow(uu, _):
                        plsc.store_scatter(
                            tr,
                            [jnp.full((_L,), uu, jnp.int32), cols],
                            jnp.zeros((_L,), jnp.float32),
                            mask=zmask,
                        )
                        return 0

                    lax.fori_loop(0, D, zrow, 0)

                return 0

            lax.fori_loop(0, K // _L, zgrp, 0)

            pending_w[t % 2] = pltpu.async_copy(
                tr, out_hbm.at[j, :, pl.ds(i0, K)], wsems[t % 2]
            )
        for p in pending_w:
            if p is not None:
                p.wait()

    return k


def kernel(inputs, lookup_table):
    V, D = lookup_table.shape
    I, N = inputs.shape
    idx = inputs.T.reshape(-1).astype(jnp.int32)
    out = _build(N, I, V, D, 512)(idx, lookup_table)
    return jnp.transpose(out, (2, 0, 1))


# speedup vs baseline: 2.1873x; 1.4049x over previous
import functools

import jax
import jax.numpy as jnp
from jax import lax
from jax.experimental import pallas as pl
from jax.experimental.pallas import tpu as pltpu
from jax.experimental.pallas import tpu_sc as plsc

_NC = 2
_NS = 16
_L = 16
_NW = _NC * _NS


@functools.lru_cache(maxsize=None)
def _build(N, I, V, D, K):
    assert D == 2 * _L
    B = N * I
    per_w = B // _NW
    assert per_w * _NW == B
    units_i = I // K
    assert units_i * K == I
    units = N * units_i
    per_w_units = units // _NW
    assert per_w_units * _NW == units
    assert per_w_units >= 4 and per_w_units % 2 == 0
    scale = jnp.float32(float(D) ** 0.5)

    mesh = plsc.VectorSubcoreMesh(core_axis_name="c", subcore_axis_name="s")

    @functools.partial(
        pl.kernel,
        mesh=mesh,
        out_type=jax.ShapeDtypeStruct((N, D, I), jnp.float32),
        compiler_params=pltpu.CompilerParams(
            needs_layout_passes=False, use_tc_tiling_on_sc=False
        ),
        scratch_types=[
            pltpu.VMEM((per_w,), jnp.int32),
            pltpu.VMEM((K, D), jnp.float32),
            pltpu.VMEM((K, D), jnp.float32),
            pltpu.VMEM((D, K + 1), jnp.float32),
            pltpu.VMEM((D, K + 1), jnp.float32),
            pltpu.SemaphoreType.DMA,
            pltpu.SemaphoreType.DMA,
            pltpu.SemaphoreType.DMA,
            pltpu.SemaphoreType.DMA,
        ],
    )
    def k(idx_hbm, table_hbm, out_hbm, idx_v, raw0, raw1, tr0, tr1,
          gsem0, gsem1, wsem0, wsem1):
        wid = lax.axis_index("s") * _NC + lax.axis_index("c")
        base_u = wid * per_w_units
        pltpu.sync_copy(idx_hbm.at[pl.ds(wid * per_w, per_w)], idx_v)

        raws = (raw0, raw1)
        trs = (tr0, tr1)
        gsems = (gsem0, gsem1)
        wsems = (wsem0, wsem1)
        lane = lax.iota(jnp.int32, _L)
        u_lo = lane
        u_hi = _L + lane

        def start_gather(t, b):
            pltpu.async_copy(
                table_hbm.at[idx_v.at[pl.ds(t * K, K)]], raws[b], gsems[b]
            )

        def wait_gather(b):
            pltpu.make_async_copy(
                table_hbm.at[idx_v.at[pl.ds(0, K)]], raws[b], gsems[b]
            ).wait()

        def start_write(t, b):
            u = base_u + t
            j = u // units_i
            i0 = (u % units_i) * K
            pltpu.async_copy(
                trs[b].at[:, pl.ds(0, K)], out_hbm.at[j, :, pl.ds(i0, K)],
                wsems[b],
            )

        def wait_write(b):
            pltpu.make_async_copy(
                trs[b].at[:, pl.ds(0, K)], out_hbm.at[0, :, pl.ds(0, K)],
                wsems[b],
            ).wait()

        def compute(t, b):
            raw = raws[b]
            tr = trs[b]

            def trow(r, _):
                rcol = jnp.full((_L,), r, jnp.int32)
                plsc.store_scatter(tr, [u_lo, rcol], raw[r, pl.ds(0, _L)] * scale)
                plsc.store_scatter(tr, [u_hi, rcol], raw[r, pl.ds(_L, _L)] * scale)
                return 0

            lax.fori_loop(0, K, trow, 0, unroll=8)

            def acc_zero(g, acc):
                iv = idx_v[pl.ds(t * K + g * _L, _L)]
                return acc | (iv == 0)

            any_zero = lax.fori_loop(
                0, K // _L, acc_zero, jnp.zeros((_L,), jnp.bool_), unroll=4
            )
            nz = plsc.all_reduce_population_count(any_zero)

            @pl.when(nz[0] > 0)
            def _():
                def zgrp(g, _):
                    iv = idx_v[pl.ds(t * K + g * _L, _L)]
                    zmask = iv == 0
                    cols = g * _L + lane

                    def zrow(uu, _):
                        plsc.store_scatter(
                            tr,
                            [jnp.full((_L,), uu, jnp.int32), cols],
                            jnp.zeros((_L,), jnp.float32),
                            mask=zmask,
                        )
                        return 0

                    lax.fori_loop(0, D, zrow, 0)
                    return 0

                lax.fori_loop(0, K // _L, zgrp, 0)

        start_gather(0, 0)
        start_gather(1, 1)
        for b in (0, 1):
            wait_gather(b)
            compute(b, b)
            start_gather(2 + b, b)
            start_write(b, b)

        def pair(i, _):
            t0 = 2 * i
            for b in (0, 1):
                t = t0 + b
                wait_gather(b)
                wait_write(b)
                compute(t, b)
                start_gather(t + 2, b)
                start_write(t, b)
            return 0

        lax.fori_loop(1, per_w_units // 2 - 1, pair, 0)

        for b in (0, 1):
            t = per_w_units - 2 + b
            wait_gather(b)
            wait_write(b)
            compute(t, b)
            start_write(t, b)
        for b in (0, 1):
            wait_write(b)

    return k


def kernel(inputs, lookup_table):
    V, D = lookup_table.shape
    I, N = inputs.shape
    idx = inputs.T.reshape(-1).astype(jnp.int32)
    out = _build(N, I, V, D, 512)(idx, lookup_table)
    return jnp.transpose(out, (2, 0, 1))
